# Initial kernel scaffold; baseline (speedup 1.0000x reference)
import jax
import jax.numpy as jnp
from jax.experimental import pallas as pl


def kernel(x, expert_weights, expert_indices, w1, w2):
    raise NotImplementedError("write your pallas kernel here")


# speedup vs baseline: 2.1961x; 2.1961x over previous
import functools

import jax
import jax.numpy as jnp
from jax import lax
from jax.experimental import pallas as pl
from jax.experimental.pallas import tpu as pltpu
from jax.experimental.pallas import tpu_sc as plsc

E = 8
K = 2
T = 2048
D = 1024
F = 1024
ROWS = T * K
BLK = 256
NB = ROWS // BLK + E
NPAD = NB * BLK

NC = 2
NS = 16
NW = NC * NS
RPW = ROWS // NW
CH = 64
NCH = RPW // CH

@functools.lru_cache(maxsize=None)
def _sc_kernels():
    mesh = plsc.VectorSubcoreMesh(core_axis_name="c", subcore_axis_name="s")

    @functools.partial(
        pl.kernel,
        mesh=mesh,
        out_type=jax.ShapeDtypeStruct((NPAD, D), jnp.float32),
        scratch_types=[
            pltpu.VMEM((CH,), jnp.int32),
            pltpu.VMEM((CH,), jnp.int32),
            pltpu.VMEM((CH, D), jnp.float32),
            pltpu.SemaphoreType.DMA,
        ],
    )
    def permute_in(x_hbm, tok_hbm, dest_hbm, xs_hbm, tok_v, dest_v, rows_v, sem):
        wid = lax.axis_index("s") * NC + lax.axis_index("c")
        base = wid * RPW
        for c in range(NCH):
            off = base + c * CH
            pltpu.sync_copy(tok_hbm.at[pl.ds(off, CH)], tok_v)
            pltpu.sync_copy(dest_hbm.at[pl.ds(off, CH)], dest_v)
            pltpu.async_copy(x_hbm.at[tok_v], rows_v, sem).wait()
            pltpu.async_copy(rows_v, xs_hbm.at[dest_v], sem).wait()

    @functools.partial(
        pl.kernel,
        mesh=mesh,
        out_type=jax.ShapeDtypeStruct((ROWS, D), jnp.float32),
        scratch_types=[
            pltpu.VMEM((CH,), jnp.int32),
            pltpu.VMEM((CH, D), jnp.float32),
            pltpu.SemaphoreType.DMA,
        ],
    )
    def permute_out(ys_hbm, dest_hbm, yf_hbm, dest_v, rows_v, sem):
        wid = lax.axis_index("s") * NC + lax.axis_index("c")
        base = wid * RPW
        for c in range(NCH):
            off = base + c * CH
            pltpu.sync_copy(dest_hbm.at[pl.ds(off, CH)], dest_v)
            pltpu.async_copy(ys_hbm.at[dest_v], rows_v, sem).wait()
            pltpu.sync_copy(rows_v, yf_hbm.at[pl.ds(off, CH)])

    return permute_in, permute_out


def _gemm_body(be_ref, xs_ref, w1_ref, w2_ref, ys_ref):
    h = jnp.maximum(
        jnp.dot(xs_ref[...], w1_ref[0], preferred_element_type=jnp.float32), 0.0
    )
    ys_ref[...] = jnp.dot(h, w2_ref[0], preferred_element_type=jnp.float32)


_grouped_gemm = pl.pallas_call(
    _gemm_body,
    grid_spec=pltpu.PrefetchScalarGridSpec(
        num_scalar_prefetch=1,
        grid=(NB,),
        in_specs=[
            pl.BlockSpec((BLK, D), lambda b, be: (b, 0)),
            pl.BlockSpec((1, D, F), lambda b, be: (be[b], 0, 0)),
            pl.BlockSpec((1, F, D), lambda b, be: (be[b], 0, 0)),
        ],
        out_specs=pl.BlockSpec((BLK, D), lambda b, be: (b, 0)),
    ),
    out_shape=jax.ShapeDtypeStruct((NPAD, D), jnp.float32),
    compiler_params=pltpu.CompilerParams(
        dimension_semantics=("arbitrary",),
    ),
)


TBC = 256


def _combine_body(yf_ref, w_ref, out_ref):
    w = w_ref[...]
    out_ref[...] = (
        yf_ref[:, 0, :] * w[:, 0][:, None] + yf_ref[:, 1, :] * w[:, 1][:, None]
    )


_combine = pl.pallas_call(
    _combine_body,
    grid=(T // TBC,),
    in_specs=[
        pl.BlockSpec((TBC, K, D), lambda i: (i, 0, 0)),
        pl.BlockSpec((TBC, K), lambda i: (i, 0)),
    ],
    out_specs=pl.BlockSpec((TBC, D), lambda i: (i, 0)),
    out_shape=jax.ShapeDtypeStruct((T, D), jnp.float32),
)


def _routing(expert_indices):
    flat_e = expert_indices.reshape(-1).astype(jnp.int32)
    onehot = (flat_e[:, None] == jnp.arange(E, dtype=jnp.int32)).astype(jnp.int32)
    cum = jnp.cumsum(onehot, axis=0)
    counts = cum[-1]
    rank = jnp.sum(onehot * cum, axis=1) - 1
    nblk = (counts + BLK - 1) // BLK
    blk_start = jnp.cumsum(nblk) - nblk
    pad_base = (blk_start * BLK).astype(jnp.int32)
    dest = jnp.sum(onehot * pad_base[None, :], axis=1) + rank
    bid = jnp.arange(NB, dtype=jnp.int32)
    block_expert = jnp.sum(
        (bid[:, None] >= blk_start[None, :]).astype(jnp.int32), axis=1
    ) - 1
    block_expert = jnp.clip(block_expert, 0, E - 1)
    return counts, dest.astype(jnp.int32), block_expert.astype(jnp.int32)


def kernel(x, expert_weights, expert_indices, w1, w2):
    counts, dest, block_expert = _routing(expert_indices)
    tok = (jnp.arange(ROWS, dtype=jnp.int32) // K).astype(jnp.int32)

    permute_in, permute_out = _sc_kernels()
    xs = permute_in(x, tok, dest)
    ys = _grouped_gemm(block_expert, xs, w1, w2)
    yf = permute_out(ys, dest)

    out = _combine(yf.reshape(T, K, D), expert_weights.astype(jnp.float32))
    return out, counts


# speedup vs baseline: 2.6615x; 1.2119x over previous
import functools

import jax
import jax.numpy as jnp
from jax import lax
from jax.experimental import pallas as pl
from jax.experimental.pallas import tpu as pltpu
from jax.experimental.pallas import tpu_sc as plsc

E = 8
K = 2
T = 2048
D = 1024
F = 1024
ROWS = T * K
BLK = 256
NB = 23
NPAD = NB * BLK

RR = 32
RL = 128

NC = 2
NS = 16
NW = NC * NS
RPW = ROWS // NW
CH = 64
NCH = RPW // CH
TPW = T // NW


def _routing_body(fe_ref, counts_ref, dest_ref, be_ref):
    fe = fe_ref[...]
    rank = jnp.zeros((RR, RL), jnp.int32)
    dest = jnp.zeros((RR, RL), jnp.int32)
    counts = jnp.zeros((1, E), jnp.int32)
    bexp = jnp.zeros((1, NB), jnp.int32)
    lane_e = lax.broadcasted_iota(jnp.int32, (1, E), 1)
    lane_b = lax.broadcasted_iota(jnp.int32, (1, NB), 1)
    blk_start = jnp.int32(0)
    for e in range(E):
        m = (fe == e).astype(jnp.int32)
        ic = m
        for s in (1, 2, 4, 8, 16, 32, 64):
            ic = ic + jnp.concatenate(
                [jnp.zeros((RR, s), jnp.int32), ic[:, : RL - s]], axis=1
            )
        rt = ic[:, RL - 1 :]
        er = rt
        for s in (1, 2, 4, 8, 16):
            er = er + jnp.concatenate(
                [jnp.zeros((s, 1), jnp.int32), er[: RR - s, :]], axis=0
            )
        er = er - rt
        c_e = er[RR - 1, 0] + rt[RR - 1, 0]
        nblk_e = (c_e + BLK - 1) // BLK
        pad_base = blk_start * BLK
        rank_e = er + ic - 1
        dest = dest + m * (rank_e + pad_base)
        counts = counts + jnp.where(lane_e == e, c_e, 0)
        bexp = bexp + (lane_b >= blk_start).astype(jnp.int32)
        blk_start = blk_start + nblk_e
    counts_ref[...] = counts
    dest_ref[...] = dest
    be_ref[...] = jnp.clip(bexp - 1, 0, E - 1)


_routing_call = pl.pallas_call(
    _routing_body,
    out_shape=[
        jax.ShapeDtypeStruct((1, E), jnp.int32),
        jax.ShapeDtypeStruct((RR, RL), jnp.int32),
        jax.ShapeDtypeStruct((1, NB), jnp.int32),
    ],
)


def _routing(expert_indices):
    fe = expert_indices.reshape(RR, RL).astype(jnp.int32)
    counts, dest, block_expert = _routing_call(fe)
    return counts.reshape(E), dest.reshape(ROWS), block_expert.reshape(NB)


@functools.lru_cache(maxsize=None)
def _sc_kernels():
    mesh = plsc.VectorSubcoreMesh(core_axis_name="c", subcore_axis_name="s")

    @functools.partial(
        pl.kernel,
        mesh=mesh,
        out_type=jax.ShapeDtypeStruct((NPAD, D), jnp.float32),
        scratch_types=[
            pltpu.VMEM((CH,), jnp.int32),
            pltpu.VMEM((CH,), jnp.int32),
            pltpu.VMEM((CH, D), jnp.float32),
            pltpu.SemaphoreType.DMA,
        ],
    )
    def permute_in(x_hbm, tok_hbm, dest_hbm, xs_hbm, tok_v, dest_v, rows_v, sem):
        wid = lax.axis_index("s") * NC + lax.axis_index("c")
        base = wid * RPW
        for c in range(NCH):
            off = base + c * CH
            pltpu.sync_copy(tok_hbm.at[pl.ds(off, CH)], tok_v)
            pltpu.sync_copy(dest_hbm.at[pl.ds(off, CH)], dest_v)
            pltpu.async_copy(x_hbm.at[tok_v], rows_v, sem).wait()
            pltpu.async_copy(rows_v, xs_hbm.at[dest_v], sem).wait()

    @functools.partial(
        pl.kernel,
        mesh=mesh,
        out_type=[
            jax.ShapeDtypeStruct((T, D), jnp.float32),
            jax.ShapeDtypeStruct((T, D), jnp.float32),
        ],
        scratch_types=[
            pltpu.VMEM((TPW,), jnp.int32),
            pltpu.VMEM((TPW, D), jnp.float32),
            pltpu.SemaphoreType.DMA,
        ],
    )
    def permute_out(ys_hbm, d0_hbm, d1_hbm, y0_hbm, y1_hbm, d_v, rows_v, sem):
        wid = lax.axis_index("s") * NC + lax.axis_index("c")
        base = wid * TPW
        pltpu.sync_copy(d0_hbm.at[pl.ds(base, TPW)], d_v)
        pltpu.async_copy(ys_hbm.at[d_v], rows_v, sem).wait()
        pltpu.sync_copy(rows_v, y0_hbm.at[pl.ds(base, TPW)])
        pltpu.sync_copy(d1_hbm.at[pl.ds(base, TPW)], d_v)
        pltpu.async_copy(ys_hbm.at[d_v], rows_v, sem).wait()
        pltpu.sync_copy(rows_v, y1_hbm.at[pl.ds(base, TPW)])

    return permute_in, permute_out


def _gemm_body(be_ref, xs_ref, w1_ref, w2_ref, ys_ref):
    h = jnp.maximum(
        jnp.dot(xs_ref[...], w1_ref[0], preferred_element_type=jnp.float32), 0.0
    )
    ys_ref[...] = jnp.dot(h, w2_ref[0], preferred_element_type=jnp.float32)


_grouped_gemm = pl.pallas_call(
    _gemm_body,
    grid_spec=pltpu.PrefetchScalarGridSpec(
        num_scalar_prefetch=1,
        grid=(NB,),
        in_specs=[
            pl.BlockSpec((BLK, D), lambda b, be: (b, 0)),
            pl.BlockSpec((1, D, F), lambda b, be: (be[b], 0, 0)),
            pl.BlockSpec((1, F, D), lambda b, be: (be[b], 0, 0)),
        ],
        out_specs=pl.BlockSpec((BLK, D), lambda b, be: (b, 0)),
    ),
    out_shape=jax.ShapeDtypeStruct((NPAD, D), jnp.float32),
    compiler_params=pltpu.CompilerParams(
        dimension_semantics=("arbitrary",),
    ),
)


TBC = 256


def _combine_body(y0_ref, y1_ref, w_ref, out_ref):
    w = w_ref[...]
    out_ref[...] = y0_ref[...] * w[:, 0][:, None] + y1_ref[...] * w[:, 1][:, None]


_combine = pl.pallas_call(
    _combine_body,
    grid=(T // TBC,),
    in_specs=[
        pl.BlockSpec((TBC, D), lambda i: (i, 0)),
        pl.BlockSpec((TBC, D), lambda i: (i, 0)),
        pl.BlockSpec((TBC, K), lambda i: (i, 0)),
    ],
    out_specs=pl.BlockSpec((TBC, D), lambda i: (i, 0)),
    out_shape=jax.ShapeDtypeStruct((T, D), jnp.float32),
)


def kernel(x, expert_weights, expert_indices, w1, w2):
    counts, dest, block_expert = _routing(expert_indices)
    tok = (jnp.arange(ROWS, dtype=jnp.int32) // K).astype(jnp.int32)
    dp = dest.reshape(T, K)
    d0 = dp[:, 0]
    d1 = dp[:, 1]

    permute_in, permute_out = _sc_kernels()
    xs = permute_in(x, tok, dest)
    ys = _grouped_gemm(block_expert, xs, w1, w2)
    y0, y1 = permute_out(ys, d0, d1)

    out = _combine(y0, y1, expert_weights.astype(jnp.float32))
    return out, counts


# speedup vs baseline: 2.7283x; 1.0251x over previous
import functools

import jax
import jax.numpy as jnp
from jax import lax
from jax.experimental import pallas as pl
from jax.experimental.pallas import tpu as pltpu
from jax.experimental.pallas import tpu_sc as plsc

E = 8
K = 2
T = 2048
D = 1024
F = 1024
ROWS = T * K
BLK = 256
NB = 23
NPAD = NB * BLK

RR = 32
RL = 128

NC = 2
NS = 16
NW = NC * NS
RPW = ROWS // NW
CH = 64
NCH = RPW // CH
TPW = T // NW


def _routing_body(fe_ref, ew_ref, counts_ref, dest_ref, be_ref, w0_ref, w1_ref):
    ew = ew_ref[...]
    w0_ref[...] = jnp.broadcast_to(ew[:, 0:1], (T, 16))
    w1_ref[...] = jnp.broadcast_to(ew[:, 1:2], (T, 16))
    fe = fe_ref[...]
    rank = jnp.zeros((RR, RL), jnp.int32)
    dest = jnp.zeros((RR, RL), jnp.int32)
    counts = jnp.zeros((1, E), jnp.int32)
    bexp = jnp.zeros((1, NB), jnp.int32)
    lane_e = lax.broadcasted_iota(jnp.int32, (1, E), 1)
    lane_b = lax.broadcasted_iota(jnp.int32, (1, NB), 1)
    blk_start = jnp.int32(0)
    for e in range(E):
        m = (fe == e).astype(jnp.int32)
        ic = m
        for s in (1, 2, 4, 8, 16, 32, 64):
            ic = ic + jnp.concatenate(
                [jnp.zeros((RR, s), jnp.int32), ic[:, : RL - s]], axis=1
            )
        rt = ic[:, RL - 1 :]
        er = rt
        for s in (1, 2, 4, 8, 16):
            er = er + jnp.concatenate(
                [jnp.zeros((s, 1), jnp.int32), er[: RR - s, :]], axis=0
            )
        er = er - rt
        c_e = er[RR - 1, 0] + rt[RR - 1, 0]
        nblk_e = (c_e + BLK - 1) // BLK
        pad_base = blk_start * BLK
        rank_e = er + ic - 1
        dest = dest + m * (rank_e + pad_base)
        counts = counts + jnp.where(lane_e == e, c_e, 0)
        bexp = bexp + (lane_b >= blk_start).astype(jnp.int32)
        blk_start = blk_start + nblk_e
    counts_ref[...] = counts
    dest_ref[...] = dest
    be_ref[...] = jnp.clip(bexp - 1, 0, E - 1)


_routing_call = pl.pallas_call(
    _routing_body,
    out_shape=[
        jax.ShapeDtypeStruct((1, E), jnp.int32),
        jax.ShapeDtypeStruct((RR, RL), jnp.int32),
        jax.ShapeDtypeStruct((1, NB), jnp.int32),
        jax.ShapeDtypeStruct((T, 16), jnp.float32),
        jax.ShapeDtypeStruct((T, 16), jnp.float32),
    ],
)


def _routing(expert_indices, expert_weights):
    fe = expert_indices.reshape(RR, RL).astype(jnp.int32)
    counts, dest, block_expert, w0rep, w1rep = _routing_call(
        fe, expert_weights.astype(jnp.float32)
    )
    return counts.reshape(E), dest.reshape(ROWS), block_expert.reshape(NB), w0rep, w1rep


HALF = TPW // 2


@functools.lru_cache(maxsize=None)
def _sc_kernels():
    mesh = plsc.VectorSubcoreMesh(core_axis_name="c", subcore_axis_name="s")

    @functools.partial(
        pl.kernel,
        mesh=mesh,
        out_type=jax.ShapeDtypeStruct((NPAD, D), jnp.float32),
        scratch_types=[
            pltpu.VMEM((TPW,), jnp.int32),
            pltpu.VMEM((TPW,), jnp.int32),
            pltpu.VMEM((TPW, D), jnp.float32),
            pltpu.SemaphoreType.DMA,
        ],
    )
    def permute_in(x_hbm, d0_hbm, d1_hbm, xs_hbm, d0_v, d1_v, xbuf, sem):
        wid = lax.axis_index("s") * NC + lax.axis_index("c")
        base = wid * TPW
        pltpu.sync_copy(d0_hbm.at[pl.ds(base, TPW)], d0_v)
        pltpu.sync_copy(d1_hbm.at[pl.ds(base, TPW)], d1_v)
        pltpu.sync_copy(x_hbm.at[pl.ds(base, TPW)], xbuf)
        c0 = pltpu.async_copy(xbuf, xs_hbm.at[d0_v], sem)
        c1 = pltpu.async_copy(xbuf, xs_hbm.at[d1_v], sem)
        c0.wait()
        c1.wait()

    @functools.partial(
        pl.kernel,
        mesh=mesh,
        out_type=jax.ShapeDtypeStruct((T, D), jnp.float32),
        scratch_types=[
            pltpu.VMEM((HALF,), jnp.int32),
            pltpu.VMEM((HALF,), jnp.int32),
            pltpu.VMEM((TPW, 16), jnp.float32),
            pltpu.VMEM((TPW, 16), jnp.float32),
            pltpu.VMEM((HALF, D), jnp.float32),
            pltpu.VMEM((HALF, D), jnp.float32),
            pltpu.VMEM((HALF, D), jnp.float32),
            pltpu.SemaphoreType.DMA,
        ],
    )
    def combine(ys_hbm, d0_hbm, d1_hbm, w0_hbm, w1_hbm, out_hbm,
                d0_v, d1_v, w0_v, w1_v, buf_a, buf_b, buf_o, sem):
        wid = lax.axis_index("s") * NC + lax.axis_index("c")
        base = wid * TPW
        pltpu.sync_copy(w0_hbm.at[pl.ds(base, TPW)], w0_v)
        pltpu.sync_copy(w1_hbm.at[pl.ds(base, TPW)], w1_v)
        for h in range(TPW // HALF):
            off = base + h * HALF
            pltpu.sync_copy(d0_hbm.at[pl.ds(off, HALF)], d0_v)
            pltpu.sync_copy(d1_hbm.at[pl.ds(off, HALF)], d1_v)
            ca = pltpu.async_copy(ys_hbm.at[d0_v], buf_a, sem)
            cb = pltpu.async_copy(ys_hbm.at[d1_v], buf_b, sem)
            ca.wait()
            cb.wait()

            def tok_body(t, _, h=h):
                w0s = w0_v[h * HALF + t, :]
                w1s = w1_v[h * HALF + t, :]
                for c in range(D // 16):
                    a = buf_a[t, pl.ds(c * 16, 16)]
                    b = buf_b[t, pl.ds(c * 16, 16)]
                    buf_o[t, pl.ds(c * 16, 16)] = w0s * a + w1s * b
                return 0

            lax.fori_loop(0, HALF, tok_body, 0)
            pltpu.sync_copy(buf_o, out_hbm.at[pl.ds(off, HALF)])

    return permute_in, combine


def _gemm_body(be_ref, xs_ref, w1_ref, w2_ref, ys_ref):
    h = jnp.maximum(
        jnp.dot(xs_ref[...], w1_ref[0], preferred_element_type=jnp.float32), 0.0
    )
    ys_ref[...] = jnp.dot(h, w2_ref[0], preferred_element_type=jnp.float32)


_grouped_gemm = pl.pallas_call(
    _gemm_body,
    grid_spec=pltpu.PrefetchScalarGridSpec(
        num_scalar_prefetch=1,
        grid=(NB,),
        in_specs=[
            pl.BlockSpec((BLK, D), lambda b, be: (b, 0)),
            pl.BlockSpec((1, D, F), lambda b, be: (be[b], 0, 0)),
            pl.BlockSpec((1, F, D), lambda b, be: (be[b], 0, 0)),
        ],
        out_specs=pl.BlockSpec((BLK, D), lambda b, be: (b, 0)),
    ),
    out_shape=jax.ShapeDtypeStruct((NPAD, D), jnp.float32),
    compiler_params=pltpu.CompilerParams(
        dimension_semantics=("arbitrary",),
    ),
)


def kernel(x, expert_weights, expert_indices, w1, w2):
    counts, dest, block_expert, w0rep, w1rep = _routing(
        expert_indices, expert_weights
    )
    dp = dest.reshape(T, K)
    d0 = dp[:, 0]
    d1 = dp[:, 1]

    permute_in, combine = _sc_kernels()
    xs = permute_in(x, d0, d1)
    ys = _grouped_gemm(block_expert, xs, w1, w2)
    out = combine(ys, d0, d1, w0rep, w1rep)
    return out, counts
